# baseline (speedup 1.0000x reference)
import functools

import jax
import jax.numpy as jnp
from jax import lax
from jax.experimental import pallas as pl
from jax.experimental.pallas import tpu as pltpu
from jax.experimental.pallas import tpu_sc as plsc

B, L, K = 4, 512, 30
NF, EF, H, NL = 10, 11, 256, 30
FF = H * 4
N = B * L
E = N * K
EPS = 1e-05

BLK = 128
GRID = N // BLK

SC_WORKERS = 32
ROWS_W = E // SC_WORKERS
CHUNK = 128
NCHUNK = ROWS_W // CHUNK
NBUF = 3


def _dot(a, b):
    return jnp.dot(a.astype(jnp.bfloat16), b.astype(jnp.bfloat16),
                   preferred_element_type=jnp.float32)


def _ln(x, g, b):
    m = jnp.mean(x, axis=-1, keepdims=True)
    d = x - m
    v = jnp.mean(d * d, axis=-1, keepdims=True)
    return d / jnp.sqrt(v + EPS) * g + b


def _sc_gather_kernel(idx_hbm, table_hbm, out_hbm, idx_v, *rest):
    bufs = rest[:NBUF]
    gsems = rest[NBUF:2 * NBUF]
    wsems = rest[2 * NBUF:]
    wid = lax.axis_index("s") * 2 + lax.axis_index("c")
    base = wid * ROWS_W
    pltpu.sync_copy(idx_hbm.at[pl.ds(base, ROWS_W)], idx_v)

    gd = [None] * NBUF
    wr = [None] * NBUF

    def _write(jp):
        p = jp % NBUF
        gd[p].wait()
        wr[p] = pltpu.async_copy(
            bufs[p], out_hbm.at[pl.ds(base + jp * CHUNK, CHUNK)], wsems[p])

    for j in range(NCHUNK):
        s = j % NBUF
        if wr[s] is not None:
            wr[s].wait()
        gd[s] = pltpu.async_copy(
            table_hbm.at[idx_v.at[pl.ds(j * CHUNK, CHUNK)]], bufs[s],
            gsems[s])
        if j >= NBUF - 1:
            _write(j - (NBUF - 1))
    for jp in range(max(0, NCHUNK - (NBUF - 1)), NCHUNK):
        _write(jp)
    for s in range(NBUF):
        if wr[s] is not None:
            wr[s].wait()


def _sc_gather(idx, table):
    mesh = plsc.VectorSubcoreMesh(core_axis_name="c", subcore_axis_name="s")
    fn = pl.kernel(
        _sc_gather_kernel,
        out_type=jax.ShapeDtypeStruct((E, H), jnp.float32),
        mesh=mesh,
        scratch_types=(
            [pltpu.VMEM((ROWS_W,), jnp.int32)]
            + [pltpu.VMEM((CHUNK, H), jnp.float32) for _ in range(NBUF)]
            + [pltpu.SemaphoreType.DMA for _ in range(2 * NBUF)]
        ),
    )
    return fn(idx, table)


def _init_kernel(nodes_ref, src_ref, wv_ref, bv_ref, hv_ref):
    x = jnp.concatenate([nodes_ref[...], src_ref[...]], axis=-1)
    hv_ref[...] = _dot(x, wv_ref[...]) + bv_ref[...]


def _layer_kernel(hv_ref, g_ref, e_ref, w1_ref, b1_ref, we_ref, be_ref,
                  w2_ref, b2_ref, w3_ref, b3_ref, ln1g_ref, ln1b_ref,
                  wf1_ref, bf1_ref, wf2_ref, bf2_ref, ln2g_ref, ln2b_ref,
                  hv_out_ref):
    hv = hv_ref[...]
    ek = e_ref[...].reshape(K * BLK, EF)
    he = _dot(ek, we_ref[...]) + be_ref[...]
    gk = g_ref[...].reshape(K * BLK, H)
    hv_t = jnp.broadcast_to(hv[None], (K, BLK, H)).reshape(K * BLK, H)
    hcat = jnp.concatenate([hv_t, he, gk], axis=-1)
    m1 = jnp.maximum(_dot(hcat, w1_ref[...]) + b1_ref[...], 0.0)
    m2 = jnp.maximum(_dot(m1, w2_ref[...]) + b2_ref[...], 0.0)
    m3 = (_dot(m2, w3_ref[...]) + b3_ref[...]).reshape(K, BLK, H)
    s = m3[0]
    for k in range(1, K):
        s = s + m3[k]
    h1 = _ln(hv + s / 30.0, ln1g_ref[...], ln1b_ref[...])
    dh = (_dot(jnp.maximum(_dot(h1, wf1_ref[...]) + bf1_ref[...], 0.0),
               wf2_ref[...]) + bf2_ref[...])
    hv_out_ref[...] = _ln(h1 + dh, ln2g_ref[...], ln2b_ref[...])


def _head1_kernel(hv_ref, pw1_ref, pb1_ref, plg_ref, plb_ref, pw2r_ref,
                  pb2_ref, out_ref):
    h = jnp.maximum(_dot(hv_ref[...], pw1_ref[...]) + pb1_ref[...], 0.0)
    h = _ln(h, plg_ref[...], plb_ref[...])
    s = _dot(h, pw2r_ref[...].reshape(2 * H, 1)) + pb2_ref[...]
    out_ref[...] = jnp.maximum(s, 0.0)


def _head2_kernel(r_ref, l1g_ref, l1b_ref, rw1_ref, rb1_ref, l2g_ref,
                  l2b_ref, rw2r_ref, rb2_ref, out_ref):
    r = _ln(r_ref[...], l1g_ref[...], l1b_ref[...])
    m = jnp.maximum(_dot(r, rw1_ref[...]) + rb1_ref[...], 0.0)
    m = _ln(m, l2g_ref[...], l2b_ref[...])
    s = _dot(m, rw2r_ref[...].reshape(2 * L, 1))
    out_ref[...] = s + rb2_ref[...] + 0.5


def _row(v):
    return v.reshape(1, -1)


def kernel(nodes, edges, src, params, connections, node_mask, lengths):
    p = params
    e_km = edges.reshape(N, K, EF).transpose(1, 0, 2)
    gidx = (connections.astype(jnp.int32)
            + (jnp.arange(B, dtype=jnp.int32) * L)[:, None, None])
    gidx_km = gidx.reshape(N, K).T.reshape(E)

    full = lambda shape: pl.BlockSpec(shape, lambda *_: (0,) * len(shape))

    hv = pl.pallas_call(
        _init_kernel,
        grid=(1,),
        in_specs=[full((N, NF)), full((N, NL)), full((NF + NL, H)),
                  full((1, H))],
        out_specs=[full((N, H))],
        out_shape=[jax.ShapeDtypeStruct((N, H), jnp.float32)],
    )(nodes.reshape(N, NF), src.reshape(N, NL), p['wv'], _row(p['bv']))[0]

    blk = lambda shape: pl.BlockSpec(shape, lambda i: (i,) + (0,) * (len(shape) - 1))
    km_blk = lambda shape: pl.BlockSpec(shape, lambda i: (0, i, 0))

    for lp in p['layers']:
        g = _sc_gather(gidx_km, hv).reshape(K, N, H)
        hv = pl.pallas_call(
            _layer_kernel,
            grid=(GRID,),
            in_specs=[blk((BLK, H)), km_blk((K, BLK, H)), km_blk((K, BLK, EF)),
                      full((3 * H, H)), full((1, H)), full((EF, H)),
                      full((1, H)),
                      full((H, H)), full((1, H)), full((H, H)), full((1, H)),
                      full((1, H)), full((1, H)),
                      full((H, FF)), full((1, FF)), full((FF, H)), full((1, H)),
                      full((1, H)), full((1, H))],
            out_specs=[blk((BLK, H))],
            out_shape=[jax.ShapeDtypeStruct((N, H), jnp.float32)],
            compiler_params=pltpu.CompilerParams(
                dimension_semantics=("arbitrary",)),
        )(hv, g, e_km,
          lp['w1'], _row(lp['b1']), p['we'], _row(p['be']),
          lp['w2'], _row(lp['b2']), lp['w3'], _row(lp['b3']),
          _row(lp['ln1g']), _row(lp['ln1b']),
          lp['wf1'], _row(lp['bf1']), lp['wf2'], _row(lp['bf2']),
          _row(lp['ln2g']), _row(lp['ln2b']))[0]

    r = pl.pallas_call(
        _head1_kernel,
        grid=(1,),
        in_specs=[full((N, H)), full((H, 2 * H)), full((1, 2 * H)),
                  full((1, 2 * H)), full((1, 2 * H)), full((1, 2 * H)),
                  full((1, 1))],
        out_specs=[full((N, 1))],
        out_shape=[jax.ShapeDtypeStruct((N, 1), jnp.float32)],
    )(hv, p['phi_w1'], _row(p['phi_b1']), _row(p['phi_lng']),
      _row(p['phi_lnb']), _row(p['phi_w2']), _row(p['phi_b2']))[0]

    out = pl.pallas_call(
        _head2_kernel,
        grid=(1,),
        in_specs=[full((B, L)), full((1, L)), full((1, L)), full((L, 2 * L)),
                  full((1, 2 * L)), full((1, 2 * L)), full((1, 2 * L)),
                  full((1, 2 * L)), full((1, 1))],
        out_specs=[full((B, 1))],
        out_shape=[jax.ShapeDtypeStruct((B, 1), jnp.float32)],
    )(r.reshape(B, L), _row(p['rho_ln1g']), _row(p['rho_ln1b']),
      p['rho_w1'], _row(p['rho_b1']), _row(p['rho_ln2g']),
      _row(p['rho_ln2b']), _row(p['rho_w2']), _row(p['rho_b2']))[0]

    return out.reshape(B)
